# baseline (speedup 1.0000x reference)
import functools

import jax
import jax.numpy as jnp
from jax import lax
from jax.experimental import pallas as pl
from jax.experimental.pallas import tpu as pltpu
from jax.experimental.pallas import tpu_sc as plsc

B = 16
ROWS = 896
COLS = 5313
BIN0 = 416
NBIN = 64
NTRK = 32
TSTRIDE = 100
OUT_N = B * NTRK


def _sc_score(flat_in):
    mesh = plsc.VectorSubcoreMesh(core_axis_name="c", subcore_axis_name="s")

    @functools.partial(
        pl.kernel,
        mesh=mesh,
        out_type=jax.ShapeDtypeStruct((OUT_N,), jnp.float32),
        scratch_types=[
            pltpu.VMEM((8, 128), jnp.int32),
            pltpu.VMEM((8, 128), jnp.float32),
            pltpu.VMEM((16,), jnp.float32),
            pltpu.SemaphoreType.DMA,
        ],
    )
    def k(in_hbm, out_hbm, idx_v, rows_v, res_v, sem):
        w = lax.axis_index("s") * 2 + lax.axis_index("c")
        b = w // 2
        t0 = (w % 2) * 16
        base = b * (ROWS * COLS) + BIN0 * COLS + t0 * TSTRIDE
        lane_off = lax.iota(jnp.int32, 16) * TSTRIDE
        for j in range(8):
            for c in range(8):
                i = j * 8 + c
                idx_v[j, pl.ds(c * 16, 16)] = lane_off + (base + i * COLS)
        copies = [
            pltpu.async_copy(in_hbm.at[idx_v.at[j]], rows_v.at[j], sem)
            for j in range(8)
        ]
        for cp in copies:
            cp.wait()
        acc = rows_v[0, pl.ds(0, 16)]
        for j in range(8):
            for c in range(8):
                if j == 0 and c == 0:
                    continue
                acc = acc + rows_v[j, pl.ds(c * 16, 16)]
        res_v[...] = acc
        pltpu.sync_copy(res_v, out_hbm.at[pl.ds(w * 16, 16)])

    return k(flat_in)


def kernel(inputs):
    flat = inputs.reshape(-1)
    out = _sc_score(flat)
    return out.reshape(B, NTRK)


# speedup vs baseline: 11.8277x; 11.8277x over previous
import functools

import jax
import jax.numpy as jnp
from jax import lax
from jax.experimental import pallas as pl
from jax.experimental.pallas import tpu as pltpu
from jax.experimental.pallas import tpu_sc as plsc

B = 16
ROWS = 896
COLS = 5313
BIN0 = 416
NBIN = 64
NTRK = 32
TSTRIDE = 100
OUT_N = B * NTRK
CW = 1664


def _sc_score(x):
    mesh = plsc.VectorSubcoreMesh(core_axis_name="c", subcore_axis_name="s")

    @functools.partial(
        pl.kernel,
        mesh=mesh,
        out_type=jax.ShapeDtypeStruct((OUT_N,), jnp.float32),
        scratch_types=[
            pltpu.VMEM((NBIN, CW), jnp.float32),
            pltpu.VMEM((16,), jnp.float32),
            pltpu.SemaphoreType.DMA,
        ],
        compiler_params=pltpu.CompilerParams(
            use_tc_tiling_on_sc=True, needs_layout_passes=False
        ),
    )
    def k(in_hbm, out_hbm, buf_v, res_v, sem):
        w = lax.axis_index("s") * 2 + lax.axis_index("c")
        b = w // 2
        h = w % 2
        c0 = h * 1536
        pltpu.async_copy(
            in_hbm.at[b, pl.ds(BIN0, NBIN), pl.ds(c0, CW)], buf_v, sem
        ).wait()
        col_idx = lax.iota(jnp.int32, 16) * TSTRIDE + h * 64
        acc = None
        for i in range(NBIN):
            row_idx = jnp.full((16,), i, dtype=jnp.int32)
            v = plsc.load_gather(buf_v, [row_idx, col_idx])
            acc = v if acc is None else acc + v
        res_v[...] = acc
        pltpu.sync_copy(res_v, out_hbm.at[pl.ds(w * 16, 16)])

    return k(x)


def kernel(inputs):
    out = _sc_score(inputs)
    return out.reshape(B, NTRK)
